# baseline (speedup 1.0000x reference)
import functools

import jax
import jax.numpy as jnp
from jax import lax
from jax.experimental import pallas as pl
from jax.experimental.pallas import tpu as pltpu
from jax.experimental.pallas import tpu_sc as plsc

B = 16384
D = 64
N_NEG = 20
BN = B * N_NEG

NC = 2
NS = 16
NW = NC * NS

IW = 128
KS = 4
SUP = IW * KS

G1_ROWS = B // NW
G3_ROWS = BN // NW
G3_SUPER = G3_ROWS // SUP


def _build_kernel():
    mesh = plsc.VectorSubcoreMesh(core_axis_name="c", subcore_axis_name="s")

    @functools.partial(
        pl.kernel,
        mesh=mesh,
        compiler_params=pltpu.CompilerParams(use_tc_tiling_on_sc=False),
        out_type=[
            jax.ShapeDtypeStruct((B, D), jnp.float32),
            jax.ShapeDtypeStruct((B, D), jnp.float32),
            jax.ShapeDtypeStruct((BN, D), jnp.float32),
        ],
        scratch_types=[
            pltpu.VMEM((KS, IW), jnp.int32),
            pltpu.VMEM((SUP, D), jnp.float32),
            pltpu.SemaphoreType.DMA,
        ],
    )
    def k(in_tok, ctx_tok, neg_tok, in_tab, out_tab, o1, o2, o3,
          idx_v, rows_v, gsem):
        wid = lax.axis_index("s") * NC + lax.axis_index("c")

        def one_super(tok2d, tab, out, r0, o0):
            pltpu.sync_copy(tok2d.at[pl.ds(r0, KS)], idx_v)
            cps = [
                pltpu.async_copy(
                    tab.at[idx_v.at[j]], rows_v.at[pl.ds(j * IW, IW)], gsem)
                for j in range(KS)
            ]
            for c in cps:
                c.wait()
            pltpu.sync_copy(rows_v, out.at[pl.ds(o0, SUP)])

        one_super(in_tok, in_tab, o1, wid * KS, wid * SUP)
        one_super(ctx_tok, out_tab, o2, wid * KS, wid * SUP)

        def body(s, carry):
            one_super(neg_tok, out_tab, o3,
                      wid * (G3_ROWS // IW) + s * KS,
                      wid * G3_ROWS + s * SUP)
            return carry

        lax.fori_loop(0, G3_SUPER, body, 0)

    return k


_gather_kernel = _build_kernel()


@jax.jit
def kernel(input_tokens, context_tokens, negative_context, in_table, out_table):
    in2d = input_tokens.reshape(B // IW, IW)
    ctx2d = context_tokens.reshape(B // IW, IW)
    neg2d = negative_context.reshape(BN // IW, IW)
    o1, o2, o3 = _gather_kernel(in2d, ctx2d, neg2d, in_table, out_table)
    return o1, o2, o3.reshape(B, N_NEG, D)


# speedup vs baseline: 1.0022x; 1.0022x over previous
import functools

import jax
import jax.numpy as jnp
from jax import lax
from jax.experimental import pallas as pl
from jax.experimental.pallas import tpu as pltpu
from jax.experimental.pallas import tpu_sc as plsc

B = 16384
D = 64
N_NEG = 20
BN = B * N_NEG
VOCAB = 1000000
VOCAB_PAD = VOCAB

NC = 2
NS = 16
NW = NC * NS

IW = 128
KS = 4
SUP = IW * KS

G1_ROWS = B // NW
G3_ROWS = BN // NW
G3_SUPER = G3_ROWS // SUP


TBLK = 2048


def _tc_transpose_body(x_ref, o_ref):
    y = x_ref[...].T.reshape(TBLK // 2, 2, D)
    o_ref[...] = jnp.concatenate([y[:, 0, :], y[:, 1, :]], axis=1)


def _tc_transpose(table_t):
    grid = (VOCAB + TBLK - 1) // TBLK
    return pl.pallas_call(
        _tc_transpose_body,
        grid=(grid,),
        in_specs=[pl.BlockSpec((D, TBLK), lambda i: (0, i))],
        out_specs=pl.BlockSpec((TBLK // 2, 128), lambda i: (i, 0)),
        out_shape=jax.ShapeDtypeStruct((VOCAB // 2, 128), jnp.float32),
    )(table_t)


def _build_kernel():
    mesh = plsc.VectorSubcoreMesh(core_axis_name="c", subcore_axis_name="s")

    @functools.partial(
        pl.kernel,
        mesh=mesh,
        compiler_params=pltpu.CompilerParams(use_tc_tiling_on_sc=False),
        out_type=[
            jax.ShapeDtypeStruct((B, D), jnp.float32),
            jax.ShapeDtypeStruct((B, D), jnp.float32),
            jax.ShapeDtypeStruct((BN, D), jnp.float32),
        ],
        scratch_types=[
            pltpu.VMEM((KS, IW), jnp.int32),
            pltpu.VMEM((SUP, D), jnp.float32),
            pltpu.SemaphoreType.DMA,
        ],
    )
    def k(in_tok, ctx_tok, neg_tok, in_tab, out_tab, o1, o2, o3,
          idx_v, rows_v, gsem):
        wid = lax.axis_index("s") * NC + lax.axis_index("c")

        def one_super(tok2d, tab, out, r0, o0):
            pltpu.sync_copy(tok2d.at[pl.ds(r0, KS)], idx_v)
            cps = [
                pltpu.async_copy(
                    tab.at[idx_v.at[j]], rows_v.at[pl.ds(j * IW, IW)], gsem)
                for j in range(KS)
            ]
            for c in cps:
                c.wait()
            pltpu.sync_copy(rows_v, out.at[pl.ds(o0, SUP)])

        one_super(in_tok, in_tab, o1, wid * KS, wid * SUP)
        one_super(ctx_tok, out_tab, o2, wid * KS, wid * SUP)

        def body(s, carry):
            one_super(neg_tok, out_tab, o3,
                      wid * (G3_ROWS // IW) + s * KS,
                      wid * G3_ROWS + s * SUP)
            return carry

        lax.fori_loop(0, G3_SUPER, body, 0)

    return k


_gather_kernel = _build_kernel()


@jax.jit
def kernel(input_tokens, context_tokens, negative_context, in_table, out_table):
    in2d = input_tokens.reshape(B // IW, IW)
    ctx2d = context_tokens.reshape(B // IW, IW)
    neg2d = negative_context.reshape(BN // IW, IW)
    it_lin = _tc_transpose(in_table.T).reshape(VOCAB, D)
    ot_lin = _tc_transpose(out_table.T).reshape(VOCAB, D)
    o1, o2, o3 = _gather_kernel(in2d, ctx2d, neg2d, it_lin, ot_lin)
    return o1, o2, o3.reshape(B, N_NEG, D)


# speedup vs baseline: 1.4318x; 1.4285x over previous
import functools

import jax
import jax.numpy as jnp
from jax import lax
from jax.experimental import pallas as pl
from jax.experimental.pallas import tpu as pltpu
from jax.experimental.pallas import tpu_sc as plsc

B = 16384
D = 64
N_NEG = 20
BN = B * N_NEG
VOCAB = 1000000
VOCAB_PAD = VOCAB

NC = 2
NS = 16
NW = NC * NS

IW = 128
KS = 4
SUP = IW * KS

G1_ROWS = B // NW
G3_ROWS = BN // NW
G3_SUPER = G3_ROWS // SUP


TBLK = 2048


def _tc_relayout_body(xa_ref, xb_ref, o_ref):
    o_ref[...] = jnp.concatenate([xa_ref[...].T, xb_ref[...].T], axis=1)


def _tc_relayout(in_table_t, out_table_t):
    grid = (VOCAB + TBLK - 1) // TBLK
    return pl.pallas_call(
        _tc_relayout_body,
        grid=(grid,),
        in_specs=[
            pl.BlockSpec((D, TBLK), lambda i: (0, i)),
            pl.BlockSpec((D, TBLK), lambda i: (0, i)),
        ],
        out_specs=pl.BlockSpec((TBLK, 2 * D), lambda i: (i, 0)),
        out_shape=jax.ShapeDtypeStruct((VOCAB, 2 * D), jnp.float32),
    )(in_table_t, out_table_t)


def _build_kernel():
    mesh = plsc.VectorSubcoreMesh(core_axis_name="c", subcore_axis_name="s")

    @functools.partial(
        pl.kernel,
        mesh=mesh,
        compiler_params=pltpu.CompilerParams(use_tc_tiling_on_sc=False),
        out_type=[
            jax.ShapeDtypeStruct((B, D), jnp.float32),
            jax.ShapeDtypeStruct((B, D), jnp.float32),
            jax.ShapeDtypeStruct((BN, D), jnp.float32),
        ],
        scratch_types=[
            pltpu.VMEM((KS, IW), jnp.int32),
            pltpu.VMEM((SUP, D), jnp.float32),
            pltpu.SemaphoreType.DMA,
        ],
    )
    def k(in_tok, ctx_tok, neg_tok, tab, o1, o2, o3,
          idx_v, rows_v, gsem):
        wid = lax.axis_index("s") * NC + lax.axis_index("c")

        def one_super(tok2d, out, r0, o0):
            pltpu.sync_copy(tok2d.at[pl.ds(r0, KS)], idx_v)
            cps = [
                pltpu.async_copy(
                    tab.at[idx_v.at[j]], rows_v.at[pl.ds(j * IW, IW)], gsem)
                for j in range(KS)
            ]
            for c in cps:
                c.wait()
            pltpu.sync_copy(rows_v, out.at[pl.ds(o0, SUP)])

        one_super(in_tok, o1, wid * KS, wid * SUP)
        one_super(ctx_tok, o2, wid * KS, wid * SUP)

        def body(s, carry):
            one_super(neg_tok, o3,
                      wid * (G3_ROWS // IW) + s * KS,
                      wid * G3_ROWS + s * SUP)
            return carry

        lax.fori_loop(0, G3_SUPER, body, 0)

    return k


_gather_kernel = _build_kernel()


@jax.jit
def kernel(input_tokens, context_tokens, negative_context, in_table, out_table):
    in2d = (2 * input_tokens).reshape(B // IW, IW)
    ctx2d = (2 * context_tokens + 1).reshape(B // IW, IW)
    neg2d = (2 * negative_context + 1).reshape(BN // IW, IW)
    comb = _tc_relayout(in_table.T, out_table.T).reshape(2 * VOCAB, D)
    o1, o2, o3 = _gather_kernel(in2d, ctx2d, neg2d, comb)
    return o1, o2, o3.reshape(B, N_NEG, D)


# speedup vs baseline: 1.6579x; 1.1579x over previous
import functools

import jax
import jax.numpy as jnp
from jax import lax
from jax.experimental import pallas as pl
from jax.experimental.pallas import tpu as pltpu
from jax.experimental.pallas import tpu_sc as plsc

B = 16384
D = 64
N_NEG = 20
BN = B * N_NEG
VOCAB = 1000000
VOCAB_PAD = VOCAB

NC = 2
NS = 16
NW = NC * NS

IW = 128
KS = 4
SUP = IW * KS

G1_ROWS = B // NW
G3_ROWS = BN // NW
G3_SUPER = G3_ROWS // SUP


TBLK = 4096


def _tc_relayout_body(xa_ref, xb_ref, o_ref):
    o_ref[:, 0:D] = xa_ref[...].T
    o_ref[:, D:2 * D] = xb_ref[...].T


def _tc_relayout(in_table_t, out_table_t):
    grid = (VOCAB + TBLK - 1) // TBLK
    return pl.pallas_call(
        _tc_relayout_body,
        grid=(grid,),
        in_specs=[
            pl.BlockSpec((D, TBLK), lambda i: (0, i)),
            pl.BlockSpec((D, TBLK), lambda i: (0, i)),
        ],
        out_specs=pl.BlockSpec((TBLK, 2 * D), lambda i: (i, 0)),
        out_shape=jax.ShapeDtypeStruct((VOCAB, 2 * D), jnp.float32),
    )(in_table_t, out_table_t)


def _build_kernel():
    mesh = plsc.VectorSubcoreMesh(core_axis_name="c", subcore_axis_name="s")

    @functools.partial(
        pl.kernel,
        mesh=mesh,
        compiler_params=pltpu.CompilerParams(use_tc_tiling_on_sc=False),
        out_type=[
            jax.ShapeDtypeStruct((B, D), jnp.float32),
            jax.ShapeDtypeStruct((B, D), jnp.float32),
            jax.ShapeDtypeStruct((BN, D), jnp.float32),
        ],
        scratch_types=[
            pltpu.VMEM((KS, IW), jnp.int32),
            pltpu.VMEM((SUP, D), jnp.float32),
            pltpu.SemaphoreType.DMA,
        ],
    )
    def k(in_tok, ctx_tok, neg_tok, tab, o1, o2, o3,
          idx_v, rows_v, gsem):
        wid = lax.axis_index("s") * NC + lax.axis_index("c")

        def one_super(tok2d, out, r0, o0):
            pltpu.sync_copy(tok2d.at[pl.ds(r0, KS)], idx_v)
            cps = [
                pltpu.async_copy(
                    tab.at[idx_v.at[j]], rows_v.at[pl.ds(j * IW, IW)], gsem)
                for j in range(KS)
            ]
            for c in cps:
                c.wait()
            pltpu.sync_copy(rows_v, out.at[pl.ds(o0, SUP)])

        one_super(in_tok, o1, wid * KS, wid * SUP)
        one_super(ctx_tok, o2, wid * KS, wid * SUP)

        def body(s, carry):
            one_super(neg_tok, o3,
                      wid * (G3_ROWS // IW) + s * KS,
                      wid * G3_ROWS + s * SUP)
            return carry

        lax.fori_loop(0, G3_SUPER, body, 0)

    return k


_gather_kernel = _build_kernel()


@jax.jit
def kernel(input_tokens, context_tokens, negative_context, in_table, out_table):
    in2d = (2 * input_tokens).reshape(B // IW, IW)
    ctx2d = (2 * context_tokens + 1).reshape(B // IW, IW)
    neg2d = (2 * negative_context + 1).reshape(BN // IW, IW)
    comb = _tc_relayout(in_table.T, out_table.T).reshape(2 * VOCAB, D)
    o1, o2, o3 = _gather_kernel(in2d, ctx2d, neg2d, comb)
    return o1, o2, o3.reshape(B, N_NEG, D)


# speedup vs baseline: 1.8013x; 1.0865x over previous
import functools

import jax
import jax.numpy as jnp
from jax import lax
from jax.experimental import pallas as pl
from jax.experimental.pallas import tpu as pltpu
from jax.experimental.pallas import tpu_sc as plsc

B = 16384
D = 64
N_NEG = 20
BN = B * N_NEG
VOCAB = 1000000
VOCAB_PAD = VOCAB

NC = 2
NS = 16
NW = NC * NS

IW = 128
KS = 4
SUP = IW * KS

G1_ROWS = B // NW
G3_ROWS = BN // NW
G3_SUPER = G3_ROWS // SUP


TBLK = 8192


def _tc_relayout_body(xa_ref, xb_ref, o_ref):
    o_ref[:, 0:D] = xa_ref[...].T
    o_ref[:, D:2 * D] = xb_ref[...].T


def _tc_relayout(in_table_t, out_table_t):
    grid = (VOCAB + TBLK - 1) // TBLK
    return pl.pallas_call(
        _tc_relayout_body,
        grid=(grid,),
        in_specs=[
            pl.BlockSpec((D, TBLK), lambda i: (0, i)),
            pl.BlockSpec((D, TBLK), lambda i: (0, i)),
        ],
        out_specs=pl.BlockSpec((TBLK, 2 * D), lambda i: (i, 0)),
        out_shape=jax.ShapeDtypeStruct((VOCAB, 2 * D), jnp.float32),
    )(in_table_t, out_table_t)


def _build_kernel():
    mesh = plsc.VectorSubcoreMesh(core_axis_name="c", subcore_axis_name="s")

    @functools.partial(
        pl.kernel,
        mesh=mesh,
        compiler_params=pltpu.CompilerParams(use_tc_tiling_on_sc=False),
        out_type=[
            jax.ShapeDtypeStruct((B, D), jnp.float32),
            jax.ShapeDtypeStruct((B, D), jnp.float32),
            jax.ShapeDtypeStruct((BN, D), jnp.float32),
        ],
        scratch_types=[
            pltpu.VMEM((KS, IW), jnp.int32),
            pltpu.VMEM((SUP, D), jnp.float32),
            pltpu.SemaphoreType.DMA,
        ],
    )
    def k(in_tok, ctx_tok, neg_tok, tab, o1, o2, o3,
          idx_v, rows_v, gsem):
        wid = lax.axis_index("s") * NC + lax.axis_index("c")

        def one_super(tok2d, out, r0, o0):
            pltpu.sync_copy(tok2d.at[pl.ds(r0, KS)], idx_v)
            cps = [
                pltpu.async_copy(
                    tab.at[idx_v.at[j]], rows_v.at[pl.ds(j * IW, IW)], gsem)
                for j in range(KS)
            ]
            for c in cps:
                c.wait()
            pltpu.sync_copy(rows_v, out.at[pl.ds(o0, SUP)])

        one_super(in_tok, o1, wid * KS, wid * SUP)
        one_super(ctx_tok, o2, wid * KS, wid * SUP)

        def body(s, carry):
            one_super(neg_tok, o3,
                      wid * (G3_ROWS // IW) + s * KS,
                      wid * G3_ROWS + s * SUP)
            return carry

        lax.fori_loop(0, G3_SUPER, body, 0)

    return k


_gather_kernel = _build_kernel()


@jax.jit
def kernel(input_tokens, context_tokens, negative_context, in_table, out_table):
    in2d = (2 * input_tokens).reshape(B // IW, IW)
    ctx2d = (2 * context_tokens + 1).reshape(B // IW, IW)
    neg2d = (2 * negative_context + 1).reshape(BN // IW, IW)
    comb = _tc_relayout(in_table.T, out_table.T).reshape(2 * VOCAB, D)
    o1, o2, o3 = _gather_kernel(in2d, ctx2d, neg2d, comb)
    return o1, o2, o3.reshape(B, N_NEG, D)


# speedup vs baseline: 1.8755x; 1.0412x over previous
import functools

import jax
import jax.numpy as jnp
from jax import lax
from jax.experimental import pallas as pl
from jax.experimental.pallas import tpu as pltpu
from jax.experimental.pallas import tpu_sc as plsc

B = 16384
D = 64
N_NEG = 20
BN = B * N_NEG
VOCAB = 1000000
VOCAB_PAD = VOCAB

NC = 2
NS = 16
NW = NC * NS

IW = 128
KS = 4
SUP = IW * KS

G1_ROWS = B // NW
G3_ROWS = BN // NW
G3_SUPER = G3_ROWS // SUP


TBLK = 16384


def _tc_relayout_body(xa_ref, xb_ref, o_ref):
    o_ref[:, 0:D] = xa_ref[...].T
    o_ref[:, D:2 * D] = xb_ref[...].T


def _tc_relayout(in_table_t, out_table_t):
    grid = (VOCAB + TBLK - 1) // TBLK
    return pl.pallas_call(
        _tc_relayout_body,
        grid=(grid,),
        in_specs=[
            pl.BlockSpec((D, TBLK), lambda i: (0, i)),
            pl.BlockSpec((D, TBLK), lambda i: (0, i)),
        ],
        out_specs=pl.BlockSpec((TBLK, 2 * D), lambda i: (i, 0)),
        out_shape=jax.ShapeDtypeStruct((VOCAB, 2 * D), jnp.float32),
    )(in_table_t, out_table_t)


def _build_kernel():
    mesh = plsc.VectorSubcoreMesh(core_axis_name="c", subcore_axis_name="s")

    @functools.partial(
        pl.kernel,
        mesh=mesh,
        compiler_params=pltpu.CompilerParams(use_tc_tiling_on_sc=False),
        out_type=[
            jax.ShapeDtypeStruct((B, D), jnp.float32),
            jax.ShapeDtypeStruct((B, D), jnp.float32),
            jax.ShapeDtypeStruct((BN, D), jnp.float32),
        ],
        scratch_types=[
            pltpu.VMEM((KS, IW), jnp.int32),
            pltpu.VMEM((SUP, D), jnp.float32),
            pltpu.SemaphoreType.DMA,
        ],
    )
    def k(in_tok, ctx_tok, neg_tok, tab, o1, o2, o3,
          idx_v, rows_v, gsem):
        wid = lax.axis_index("s") * NC + lax.axis_index("c")

        def one_super(tok2d, out, r0, o0):
            pltpu.sync_copy(tok2d.at[pl.ds(r0, KS)], idx_v)
            cps = [
                pltpu.async_copy(
                    tab.at[idx_v.at[j]], rows_v.at[pl.ds(j * IW, IW)], gsem)
                for j in range(KS)
            ]
            for c in cps:
                c.wait()
            pltpu.sync_copy(rows_v, out.at[pl.ds(o0, SUP)])

        one_super(in_tok, o1, wid * KS, wid * SUP)
        one_super(ctx_tok, o2, wid * KS, wid * SUP)

        def body(s, carry):
            one_super(neg_tok, o3,
                      wid * (G3_ROWS // IW) + s * KS,
                      wid * G3_ROWS + s * SUP)
            return carry

        lax.fori_loop(0, G3_SUPER, body, 0)

    return k


_gather_kernel = _build_kernel()


@jax.jit
def kernel(input_tokens, context_tokens, negative_context, in_table, out_table):
    in2d = (2 * input_tokens).reshape(B // IW, IW)
    ctx2d = (2 * context_tokens + 1).reshape(B // IW, IW)
    neg2d = (2 * negative_context + 1).reshape(BN // IW, IW)
    comb = _tc_relayout(in_table.T, out_table.T).reshape(2 * VOCAB, D)
    o1, o2, o3 = _gather_kernel(in2d, ctx2d, neg2d, comb)
    return o1, o2, o3.reshape(B, N_NEG, D)
